# Initial kernel scaffold; baseline (speedup 1.0000x reference)
import jax
import jax.numpy as jnp
from jax.experimental import pallas as pl


def kernel(pos, batch, box):
    raise NotImplementedError("write your pallas kernel here")


# speedup vs baseline: 31.6123x; 31.6123x over previous
import functools

import jax
import jax.numpy as jnp
from jax.experimental import pallas as pl
from jax.experimental.pallas import tpu as pltpu
from jax.experimental.pallas import tpu_sc as plsc

_CUTOFF2 = 25.0
_K = 64
_N = 1024
_R = 128
_BIGI = 1 << 30

_NW = 32
_EPW = (_N * _K) // _NW
_CH = _EPW // 16


def _rne_bf16(x):
    u = jax.lax.bitcast_convert_type(x, jnp.uint32)
    lsb = (u >> 16) & jnp.uint32(1)
    r = u + jnp.uint32(0x7FFF) + lsb
    return jax.lax.bitcast_convert_type(r & jnp.uint32(0xFFFF0000), jnp.float32)


def _tc_body(posT_full, batch_full, box_smem, posT_rows, batch_rows, src_ref):
    b = pl.program_id(0)
    inf = jnp.float32(jnp.inf)

    y2_t = []
    dot_t = []
    icode = []
    for d in range(3):
        pj = posT_full[d, :].reshape(1, _N)
        pi = posT_rows[d, :].reshape(_R, 1)
        boxd = box_smem[0, d]
        delta = pi - pj
        s = jnp.round(delta / boxd)
        q = pj + s * boxd
        y2_t.append(q * q)
        a = _rne_bf16(jnp.broadcast_to(pi, (_R, _N)))
        bq = _rne_bf16(q)
        dot_t.append(a * bq)
        icode.append(jnp.where(s == 0.0, 0, jnp.where(s < 0.0, 1, 2)).astype(jnp.int32))
    y2 = (y2_t[0] + y2_t[1]) + y2_t[2]
    dot = (dot_t[1] + dot_t[2]) + dot_t[0]

    pi0 = posT_rows[0, :].reshape(_R, 1)
    pi1 = posT_rows[1, :].reshape(_R, 1)
    pi2 = posT_rows[2, :].reshape(_R, 1)
    x2 = (pi0 * pi0 + pi2 * pi2) + pi1 * pi1

    d2 = x2 + y2 - 2.0 * dot
    d2 = jnp.maximum(d2, 0.0)

    bi = batch_rows[0, :].reshape(_R, 1)
    bj = batch_full[0, :].reshape(1, _N)
    mask = (d2 <= _CUTOFF2) & (bi == bj)
    keys0 = jnp.where(mask, d2, inf)

    jota = jax.lax.broadcasted_iota(jnp.int32, (_R, _N), 1)
    gidx = ((icode[0] * 3 + icode[1]) * 3 + icode[2]) * _N + jota
    tcols = jax.lax.broadcasted_iota(jnp.int32, (_R, _K), 1)

    cnt = jnp.sum(mask.astype(jnp.int32), axis=1)
    trips = jnp.minimum(jnp.max(cnt), _K)

    def round_t(t, carry):
        keys, a_src, a_key = carry
        m = jnp.min(keys, axis=1, keepdims=True)
        sel = keys == m
        gstar = jnp.min(jnp.where(sel, gidx, _BIGI), axis=1, keepdims=True)
        onehot = gidx == gstar
        jstar = gstar & (_N - 1)
        keys = jnp.where(onehot, inf, keys)
        at_t = tcols == t
        a_src = jnp.where(at_t, jstar, a_src)
        a_key = jnp.where(at_t, m, a_key)
        return keys, a_src, a_key

    carry0 = (keys0,
              jnp.zeros((_R, _K), jnp.int32),
              jnp.full((_R, _K), inf, jnp.float32))
    _, a_src, a_key = jax.lax.fori_loop(0, trips, round_t, carry0)

    row_id = b * _R + jax.lax.broadcasted_iota(jnp.int32, (_R, _K), 0)
    valid = a_key < inf
    src_ref[...] = jnp.where(valid, a_src, row_id)


def _sc_body(px_hbm, py_hbm, pz_hbm, src_hbm, dst_hbm, pdx_hbm, pdy_hbm, pdz_hbm,
             vx_hbm, vy_hbm, vz_hbm, wt_hbm,
             idxs, dsts, gx, gy, gz, hx, hy, hz, ovx, ovy, ovz, owt, sem):
    c = jax.lax.axis_index("c")
    s = jax.lax.axis_index("s")
    wid = s * 2 + c
    base = wid * _EPW
    pltpu.sync_copy(src_hbm.at[pl.ds(base, _EPW)], idxs)
    pltpu.sync_copy(dst_hbm.at[pl.ds(base, _EPW)], dsts)
    pltpu.sync_copy(pdx_hbm.at[pl.ds(base, _EPW)], hx)
    pltpu.sync_copy(pdy_hbm.at[pl.ds(base, _EPW)], hy)
    pltpu.sync_copy(pdz_hbm.at[pl.ds(base, _EPW)], hz)

    for k in range(_EPW // 128):
        isl = pl.ds(k * 128, 128)
        cps = [
            pltpu.async_copy(px_hbm.at[idxs.at[isl]], gx.at[isl], sem),
            pltpu.async_copy(py_hbm.at[idxs.at[isl]], gy.at[isl], sem),
            pltpu.async_copy(pz_hbm.at[idxs.at[isl]], gz.at[isl], sem),
        ]
        for cp in cps:
            cp.wait()

    def chunk(i, carry):
        sl = pl.ds(i * 16, 16)
        isrc = idxs[sl]
        idst = dsts[sl]
        dx = gx[sl] - hx[sl]
        dy = gy[sl] - hy[sl]
        dz = gz[sl] - hz[sl]
        nonself = isrc != idst
        sq = (dx * dx + dz * dz) + dy * dy
        s_eff = jnp.where(nonself, sq, jnp.float32(1.0))
        sb = jax.lax.bitcast_convert_type(s_eff, jnp.int32)
        x = jax.lax.bitcast_convert_type((sb >> 1) + jnp.int32(0x1FBD1DF5), jnp.float32)
        for _ in range(4):
            x = 0.5 * (x + s_eff / x)
        wt = jnp.where(nonself, x, jnp.float32(0.0))
        ovx[sl] = dx
        ovy[sl] = dy
        ovz[sl] = dz
        owt[sl] = wt
        return carry

    jax.lax.fori_loop(0, _CH, chunk, 0)

    pltpu.sync_copy(ovx, vx_hbm.at[pl.ds(base, _EPW)])
    pltpu.sync_copy(ovy, vy_hbm.at[pl.ds(base, _EPW)])
    pltpu.sync_copy(ovz, vz_hbm.at[pl.ds(base, _EPW)])
    pltpu.sync_copy(owt, wt_hbm.at[pl.ds(base, _EPW)])


@jax.jit
def kernel(pos, batch, box):
    n = pos.shape[0]
    posT = pos.T
    batch2 = batch.reshape(1, n)
    box2 = box.reshape(1, 3)

    grid = (n // _R,)
    src = pl.pallas_call(
        _tc_body,
        grid=grid,
        in_specs=[
            pl.BlockSpec((3, n), lambda b: (0, 0)),
            pl.BlockSpec((1, n), lambda b: (0, 0)),
            pl.BlockSpec(memory_space=pltpu.SMEM),
            pl.BlockSpec((3, _R), lambda b: (0, b)),
            pl.BlockSpec((1, _R), lambda b: (0, b)),
        ],
        out_specs=pl.BlockSpec((_R, _K), lambda b: (b, 0)),
        out_shape=jax.ShapeDtypeStruct((n, _K), jnp.int32),
    )(posT, batch2, box2, posT, batch2)

    src_flat = src.reshape(-1)
    px = posT[0]
    py = posT[1]
    pz = posT[2]
    col = jnp.broadcast_to(jnp.arange(n, dtype=jnp.int32)[:, None], (n, _K)).reshape(-1)
    pdx = jnp.broadcast_to(px[:, None], (n, _K)).reshape(-1)
    pdy = jnp.broadcast_to(py[:, None], (n, _K)).reshape(-1)
    pdz = jnp.broadcast_to(pz[:, None], (n, _K)).reshape(-1)

    mesh = plsc.VectorSubcoreMesh(core_axis_name="c", subcore_axis_name="s")
    sc = functools.partial(
        pl.kernel,
        mesh=mesh,
        out_type=[jax.ShapeDtypeStruct((n * _K,), jnp.float32)] * 4,
        scratch_types=(
            [pltpu.VMEM((_EPW,), jnp.int32)] * 2
            + [pltpu.VMEM((_EPW,), jnp.float32)] * 10
            + [pltpu.SemaphoreType.DMA]
        ),
    )(_sc_body)
    vx, vy, vz, wt = sc(px, py, pz, src_flat, col, pdx, pdy, pdz)

    edge_index = jnp.stack([src_flat, col])
    edge_vec = jnp.stack([vx, vy, vz], axis=-1)
    return edge_index, wt, edge_vec
